# baseline (speedup 1.0000x reference)
import functools

import jax
import jax.numpy as jnp
from jax import lax
from jax.experimental import pallas as pl
from jax.experimental.pallas import tpu as pltpu
from jax.experimental.pallas import tpu_sc as plsc

_D = 128
_NC, _NS = 2, 16
_NW = _NC * _NS
_CHUNK = 128
_NBUF = 5


def _sc_add_pe(xf, pos, pe):
    n = pos.shape[0]
    rows_per_tile = n // _NW
    n_chunks = rows_per_tile // _CHUNK

    mesh = plsc.VectorSubcoreMesh(
        core_axis_name="c", subcore_axis_name="s",
        num_cores=_NC, num_subcores=_NS)

    @functools.partial(
        pl.kernel,
        out_type=jax.ShapeDtypeStruct((n, _D), jnp.float32),
        mesh=mesh,
        compiler_params=pltpu.CompilerParams(needs_layout_passes=False),
        scratch_types=[
            [pltpu.VMEM((_CHUNK, _D), jnp.float32) for _ in range(_NBUF)],
            pltpu.VMEM((rows_per_tile,), jnp.int32),
            [pltpu.SemaphoreType.DMA for _ in range(_NBUF)],
            [pltpu.SemaphoreType.DMA for _ in range(_NBUF)],
            [pltpu.SemaphoreType.DMA for _ in range(_NBUF)],
            pltpu.VMEM_SHARED((365, _D), jnp.float32),
            pltpu.SemaphoreType.DMA,
        ],
    )
    def k(x_hbm, pos_hbm, pe_hbm, out_hbm,
          bufs, pos_v, isems, gsems, osems, pe_sh, st_sem):
        wid = lax.axis_index("s") * _NC + lax.axis_index("c")
        base = wid * rows_per_tile

        @pl.when(lax.axis_index("s") == 0)
        def _():
            pltpu.async_copy(pe_hbm, pe_sh, st_sem).wait()

        pltpu.sync_copy(pos_hbm.at[pl.ds(base, rows_per_tile)], pos_v)
        plsc.subcore_barrier()

        def start_in(c, b):
            pltpu.make_async_copy(
                x_hbm.at[pl.ds(base + c * _CHUNK, _CHUNK)],
                bufs[b], isems[b]).start()

        def wait_in(b):
            pltpu.make_async_copy(
                x_hbm.at[pl.ds(base, _CHUNK)], bufs[b], isems[b]).wait()

        def start_gather(c, b):
            pltpu.async_copy(
                pe_sh.at[pos_v.at[pl.ds(c * _CHUNK, _CHUNK)]],
                bufs[b], gsems[b], add=True)

        def wait_gather(b):
            pltpu.make_async_copy(
                pe_sh.at[pos_v.at[pl.ds(0, _CHUNK)]], bufs[b], gsems[b]).wait()

        def wait_out(b):
            pltpu.make_async_copy(
                bufs[b], out_hbm.at[pl.ds(base, _CHUNK)], osems[b]).wait()

        for b in range(_NBUF):
            start_in(b, b)
        for c in range(2):
            wait_in(c)
            start_gather(c, c)

        def body(c5, carry):
            for b in range(_NBUF):
                c = c5 * _NBUF + b
                bn = (b + 2) % _NBUF

                @pl.when(c + 2 < n_chunks)
                def _():
                    wait_in(bn)
                    start_gather(c + 2, bn)

                wait_gather(b)
                pltpu.make_async_copy(
                    bufs[b],
                    out_hbm.at[pl.ds(base + c * _CHUNK, _CHUNK)],
                    osems[b]).start()

                br = (b + _NBUF - 1) % _NBUF

                @pl.when((c >= 1) & (c + _NBUF - 1 < n_chunks))
                def _():
                    wait_out(br)
                    start_in(c + _NBUF - 1, br)
            return carry

        lax.fori_loop(0, n_chunks // _NBUF, body, 0)
        for b in range(_NBUF):
            wait_out(b)

    return k(xf, pos, pe)


def kernel(x, positions, pe):
    b, s, d = x.shape
    out = _sc_add_pe(x.reshape(b * s, d), positions.reshape(b * s), pe)
    return out.reshape(b, s, d)
